# Initial kernel scaffold; baseline (speedup 1.0000x reference)
import jax
import jax.numpy as jnp
from jax.experimental import pallas as pl


def kernel(dense_features, sparse_features, dense_W1, dense_b1, dense_W2, dense_b2, tables, out_W1, out_b1, out_W2, out_b2):
    raise NotImplementedError("write your pallas kernel here")


# speedup vs baseline: 1.3202x; 1.3202x over previous
import functools

import jax
import jax.numpy as jnp
from jax import lax
from jax.experimental import pallas as pl
from jax.experimental.pallas import tpu as pltpu
from jax.experimental.pallas import tpu_sc as plsc

B, T, V, D, DD, L, H = 4096, 26, 100000, 64, 128, 20, 512

NC, NS = 2, 16
NW = NC * NS
NB = B * T
BAGS_PER_W = NB // NW
CHUNK_BAGS = 32
IDX_PER_CHUNK = CHUNK_BAGS * L
IDX_ROWS = IDX_PER_CHUNK // 128
N_CHUNKS = BAGS_PER_W // CHUNK_BAGS


def _emb_bag_sc(tab_flat, idx_flat):
    mesh = plsc.VectorSubcoreMesh(core_axis_name="c", subcore_axis_name="s")

    @functools.partial(
        pl.kernel,
        mesh=mesh,
        out_type=jax.ShapeDtypeStruct((NB, D), jnp.float32),
        scratch_types=[
            pltpu.VMEM((IDX_PER_CHUNK,), jnp.int32),
            pltpu.VMEM((IDX_PER_CHUNK, D), jnp.float32),
            pltpu.VMEM((CHUNK_BAGS, D), jnp.float32),
            pltpu.SemaphoreType.DMA,
        ],
        compiler_params=pltpu.CompilerParams(use_tc_tiling_on_sc=False),
    )
    def emb_kernel(tab_hbm, idx_hbm, out_hbm, idx_v, rows_v, out_v, sem):
        wid = lax.axis_index("s") * NC + lax.axis_index("c")

        @pl.loop(0, N_CHUNKS)
        def _(c):
            bag0 = wid * BAGS_PER_W + c * CHUNK_BAGS
            pltpu.sync_copy(idx_hbm.at[pl.ds(bag0 * L, IDX_PER_CHUNK)], idx_v)
            handles = []
            for j in range(IDX_ROWS):
                handles.append(
                    pltpu.async_copy(
                        tab_hbm.at[idx_v.at[pl.ds(j * 128, 128)]],
                        rows_v.at[pl.ds(j * 128, 128)],
                        sem,
                    )
                )
            for h in handles:
                h.wait()

            @pl.loop(0, CHUNK_BAGS)
            def _(w):
                base = w * L
                for d in range(D // 16):
                    sl = pl.ds(d * 16, 16)

                    def body(i, acc):
                        return acc + rows_v[base + i, sl]

                    out_v[w, sl] = lax.fori_loop(
                        0, L, body, jnp.zeros((16,), jnp.float32)
                    )

            pltpu.sync_copy(out_v, out_hbm.at[pl.ds(bag0, CHUNK_BAGS)])

    return emb_kernel(tab_flat, idx_flat)


BLK = 512


def _mlp_tc(dense_features, emb_flat, dense_W1, dense_b1, dense_W2, dense_b2,
            W1d, W1e, out_b1, out_W2, out_b2):
    def body(df, emb, dW1, db1, dW2, db2, w1d, w1e, ob1, oW2, ob2, out):
        cdims = (((1,), (1,)), ((), ()))
        h1 = lax.dot_general(df[...], dW1[...], cdims,
                             preferred_element_type=jnp.float32)
        h1 = jnp.maximum(h1 + db1[...], 0.0)
        dx = lax.dot_general(h1, dW2[...], cdims,
                             preferred_element_type=jnp.float32)
        dx = jnp.maximum(dx + db2[...], 0.0)
        a = lax.dot_general(dx, w1d[...], cdims,
                            preferred_element_type=jnp.float32)
        a = a + lax.dot_general(emb[...], w1e[...], cdims,
                                preferred_element_type=jnp.float32)
        h = jnp.maximum(a + ob1[...], 0.0)
        o = lax.dot_general(h, oW2[...], cdims,
                            preferred_element_type=jnp.float32)
        out[...] = jnp.maximum(o + ob2[0, 0], 0.0)

    F_E = T * D
    whole = lambda shape: pl.BlockSpec(shape, lambda i: (0, 0))
    return pl.pallas_call(
        body,
        grid=(B // BLK,),
        in_specs=[
            pl.BlockSpec((BLK, DD), lambda i: (i, 0)),
            pl.BlockSpec((BLK, F_E), lambda i: (i, 0)),
            whole((DD, DD)),
            whole((1, DD)),
            whole((D, DD)),
            whole((1, D)),
            whole((H, D)),
            whole((H, F_E)),
            whole((1, H)),
            whole((128, H)),
            whole((1, 1)),
        ],
        out_specs=pl.BlockSpec((BLK, 128), lambda i: (i, 0)),
        out_shape=jax.ShapeDtypeStruct((B, 128), jnp.float32),
    )(dense_features, emb_flat, dense_W1, dense_b1.reshape(1, DD),
      dense_W2, dense_b2.reshape(1, D), W1d, W1e, out_b1.reshape(1, H),
      jnp.zeros((128, H), jnp.float32).at[0].set(out_W2[0]),
      out_b2.reshape(1, 1))[:, :1]


def kernel(dense_features, sparse_features, dense_W1, dense_b1, dense_W2,
           dense_b2, tables, out_W1, out_b1, out_W2, out_b2):
    assert sparse_features.shape == (T, B, L)
    assert tables.shape == (T, V, D)

    idx = jnp.transpose(sparse_features.astype(jnp.int32), (1, 0, 2))
    idx = idx + (jnp.arange(T, dtype=jnp.int32) * V)[None, :, None]
    idx_flat = idx.reshape(-1)

    tab_flat = tables.reshape(T * V, D)
    emb = _emb_bag_sc(tab_flat, idx_flat)
    emb_flat = emb.reshape(B, T * D)

    W1d = out_W1[:, :D]
    W1e = out_W1[:, D:]
    return _mlp_tc(dense_features, emb_flat, dense_W1, dense_b1, dense_W2,
                   dense_b2, W1d, W1e, out_b1, out_W2, out_b2)
